# baseline (speedup 1.0000x reference)
import functools

import jax
import jax.numpy as jnp
from jax import lax
from jax.experimental import pallas as pl
from jax.experimental.pallas import tpu as pltpu
from jax.experimental.pallas import tpu_sc as plsc

_L = 16
_NW = 32
_NSLOT = 3


def _sc_add(B, S, D):
    rows_per_w = S // _NW
    R = 8
    n_chunks = rows_per_w // R

    mesh = plsc.VectorSubcoreMesh(core_axis_name="c", subcore_axis_name="s")

    @functools.partial(
        pl.kernel,
        out_type=jax.ShapeDtypeStruct((B, S, D), jnp.float32),
        scratch_types=[
            [[pltpu.VMEM((R, D), jnp.float32) for _ in range(B)]
             for _ in range(_NSLOT)],
            [pltpu.VMEM((R, D), jnp.float32) for _ in range(_NSLOT)],
            [pltpu.SemaphoreType.DMA for _ in range(_NSLOT)],
            [pltpu.SemaphoreType.DMA for _ in range(_NSLOT)],
        ],
        mesh=mesh,
    )
    def k(x_hbm, emb_hbm, out_hbm, xslots, eslots, sins, souts):
        wid = lax.axis_index("s") * 2 + lax.axis_index("c")
        base = wid * rows_per_w

        def copies(c):
            sl = c % _NSLOT
            row = base + c * R
            ins = [(emb_hbm.at[pl.ds(row, R)], eslots[sl], sins[sl])]
            outs = []
            for b in range(B):
                ins.append((x_hbm.at[b, pl.ds(row, R)], xslots[sl][b],
                            sins[sl]))
                outs.append((xslots[sl][b], out_hbm.at[b, pl.ds(row, R)],
                             souts[sl]))
            return ins, outs

        def start_ins(c):
            for src, dst, sem in copies(c)[0]:
                pltpu.async_copy(src, dst, sem)

        def wait_ins(c):
            for src, dst, sem in copies(c)[0]:
                pltpu.make_async_copy(src, dst, sem).wait()

        def start_outs(c):
            for src, dst, sem in copies(c)[1]:
                pltpu.async_copy(src, dst, sem)

        def wait_outs(c):
            for src, dst, sem in copies(c)[1]:
                pltpu.make_async_copy(src, dst, sem).wait()

        for c in range(min(_NSLOT - 1, n_chunks)):
            start_ins(c)

        for t in range(n_chunks):
            if t + 1 < n_chunks and t + 1 >= _NSLOT - 1:
                if t >= 2:
                    wait_outs(t - 2)
                start_ins(t + 1)
            wait_ins(t)
            sl = t % _NSLOT
            xb, eb = xslots[sl], eslots[sl]

            def vec_body(i, carry, xb=xb, eb=eb):
                csl = pl.ds(i * _L, _L)
                for r in range(R):
                    ev = eb[r, csl]
                    for b in range(B):
                        xb[b][r, csl] = xb[b][r, csl] + ev
                return carry

            lax.fori_loop(0, D // _L, vec_body, 0)
            start_outs(t)

        for t in range(max(0, n_chunks - 3), n_chunks):
            wait_outs(t)

    return k


def _tc_body(x_ref, emb_ref, o_ref):
    o_ref[...] = x_ref[...] + emb_ref[...][None, :, :]


def _tc_add(x, emb, BS=512):
    Bt, S, D = x.shape
    grid = (S // BS, Bt)
    return pl.pallas_call(
        _tc_body,
        grid=grid,
        in_specs=[
            pl.BlockSpec((1, BS, D), lambda s, b: (b, s, 0)),
            pl.BlockSpec((BS, D), lambda s, b: (s, 0)),
        ],
        out_specs=pl.BlockSpec((1, BS, D), lambda s, b: (b, s, 0)),
        out_shape=jax.ShapeDtypeStruct(x.shape, x.dtype),
    )(x, emb)


def kernel(x, emb):
    B, S, D = x.shape
    e = emb[:S]
    out_tc = _tc_add(x[: B - 1], e)
    out_sc = _sc_add(1, S, D)(x[B - 1:], e)
    return jnp.concatenate([out_tc, out_sc], axis=0)


# speedup vs baseline: 1.2908x; 1.2908x over previous
import functools

import jax
import jax.numpy as jnp
from jax import lax
from jax.experimental import pallas as pl
from jax.experimental.pallas import tpu as pltpu
from jax.experimental.pallas import tpu_sc as plsc

_L = 16
_NW = 32
_NSLOT = 3


def _sc_add(B, S, D):
    rows_per_w = S // _NW
    R = 8
    n_chunks = rows_per_w // R

    mesh = plsc.VectorSubcoreMesh(core_axis_name="c", subcore_axis_name="s")

    @functools.partial(
        pl.kernel,
        out_type=jax.ShapeDtypeStruct((B, S, D), jnp.float32),
        scratch_types=[
            [[pltpu.VMEM((R, D), jnp.float32) for _ in range(B)]
             for _ in range(_NSLOT)],
            [pltpu.VMEM((R, D), jnp.float32) for _ in range(_NSLOT)],
            [pltpu.SemaphoreType.DMA for _ in range(_NSLOT)],
            [pltpu.SemaphoreType.DMA for _ in range(_NSLOT)],
        ],
        mesh=mesh,
    )
    def k(x_hbm, emb_hbm, out_hbm, xslots, eslots, sins, souts):
        wid = lax.axis_index("s") * 2 + lax.axis_index("c")
        base = wid * rows_per_w

        def copies(c):
            sl = c % _NSLOT
            row = base + c * R
            ins = [(emb_hbm.at[pl.ds(row, R)], eslots[sl], sins[sl])]
            outs = []
            for b in range(B):
                ins.append((x_hbm.at[b, pl.ds(row, R)], xslots[sl][b],
                            sins[sl]))
                outs.append((xslots[sl][b], out_hbm.at[b, pl.ds(row, R)],
                             souts[sl]))
            return ins, outs

        def start_ins(c):
            for src, dst, sem in copies(c)[0]:
                pltpu.async_copy(src, dst, sem)

        def wait_ins(c):
            for src, dst, sem in copies(c)[0]:
                pltpu.make_async_copy(src, dst, sem).wait()

        def start_outs(c):
            for src, dst, sem in copies(c)[1]:
                pltpu.async_copy(src, dst, sem)

        def wait_outs(c):
            for src, dst, sem in copies(c)[1]:
                pltpu.make_async_copy(src, dst, sem).wait()

        for c in range(min(_NSLOT - 1, n_chunks)):
            start_ins(c)

        for t in range(n_chunks):
            if t + 1 < n_chunks and t + 1 >= _NSLOT - 1:
                if t >= 2:
                    wait_outs(t - 2)
                start_ins(t + 1)
            wait_ins(t)
            sl = t % _NSLOT
            xb, eb = xslots[sl], eslots[sl]

            def vec_body(i, carry, xb=xb, eb=eb):
                csl = pl.ds(i * _L, _L)
                for r in range(R):
                    ev = eb[r, csl]
                    for b in range(B):
                        xb[b][r, csl] = xb[b][r, csl] + ev
                return carry

            lax.fori_loop(0, D // _L, vec_body, 0)
            start_outs(t)

        for t in range(max(0, n_chunks - 3), n_chunks):
            wait_outs(t)

    return k


def _tc_body(x_ref, emb_ref, o_ref):
    o_ref[...] = x_ref[...] + emb_ref[...][None, :, :]


def _tc_add(x, emb, BS=512):
    Bt, S, D = x.shape
    grid = (S // BS, Bt)
    return pl.pallas_call(
        _tc_body,
        grid=grid,
        in_specs=[
            pl.BlockSpec((1, BS, D), lambda s, b: (b, s, 0)),
            pl.BlockSpec((BS, D), lambda s, b: (s, 0)),
        ],
        out_specs=pl.BlockSpec((1, BS, D), lambda s, b: (b, s, 0)),
        out_shape=jax.ShapeDtypeStruct(x.shape, x.dtype),
    )(x, emb)


def kernel(x, emb):
    B, S, D = x.shape
    e = emb[:S]
    out_sc = _sc_add(1, S, D)(x[B - 1:], e)
    out_tc = x[: B - 1] + e[None, :, :]
    return jnp.concatenate([out_tc, out_sc], axis=0)


# speedup vs baseline: 2.0141x; 1.5603x over previous
import functools

import jax
import jax.numpy as jnp
from jax import lax
from jax.experimental import pallas as pl
from jax.experimental.pallas import tpu as pltpu
from jax.experimental.pallas import tpu_sc as plsc

_L = 16
_NW = 32
_NSLOT = 3


def _sc_add(B, S, D):
    rows_per_w = S // _NW
    R = 8
    n_chunks = rows_per_w // R

    mesh = plsc.VectorSubcoreMesh(core_axis_name="c", subcore_axis_name="s")

    @functools.partial(
        pl.kernel,
        out_type=jax.ShapeDtypeStruct((B, S, D), jnp.float32),
        scratch_types=[
            [[pltpu.VMEM((R, D), jnp.float32) for _ in range(B)]
             for _ in range(_NSLOT)],
            [pltpu.VMEM((R, D), jnp.float32) for _ in range(_NSLOT)],
            [pltpu.SemaphoreType.DMA for _ in range(_NSLOT)],
            [pltpu.SemaphoreType.DMA for _ in range(_NSLOT)],
        ],
        mesh=mesh,
    )
    def k(x_hbm, emb_hbm, out_hbm, xslots, eslots, sins, souts):
        wid = lax.axis_index("s") * 2 + lax.axis_index("c")
        base = wid * rows_per_w

        def copies(c):
            sl = c % _NSLOT
            row = base + c * R
            ins = [(emb_hbm.at[pl.ds(row, R)], eslots[sl], sins[sl])]
            outs = []
            for b in range(B):
                ins.append((x_hbm.at[b, pl.ds(row, R)], xslots[sl][b],
                            sins[sl]))
                outs.append((xslots[sl][b], out_hbm.at[b, pl.ds(row, R)],
                             souts[sl]))
            return ins, outs

        def start_ins(c):
            for src, dst, sem in copies(c)[0]:
                pltpu.async_copy(src, dst, sem)

        def wait_ins(c):
            for src, dst, sem in copies(c)[0]:
                pltpu.make_async_copy(src, dst, sem).wait()

        def start_outs(c):
            for src, dst, sem in copies(c)[1]:
                pltpu.async_copy(src, dst, sem)

        def wait_outs(c):
            for src, dst, sem in copies(c)[1]:
                pltpu.make_async_copy(src, dst, sem).wait()

        for c in range(min(_NSLOT - 1, n_chunks)):
            start_ins(c)

        for t in range(n_chunks):
            if t + 1 < n_chunks and t + 1 >= _NSLOT - 1:
                if t >= 2:
                    wait_outs(t - 2)
                start_ins(t + 1)
            wait_ins(t)
            sl = t % _NSLOT
            xb, eb = xslots[sl], eslots[sl]

            def vec_body(i, carry, xb=xb, eb=eb):
                csl = pl.ds(i * _L, _L)
                for r in range(R):
                    ev = eb[r, csl]
                    for b in range(B):
                        xb[b][r, csl] = xb[b][r, csl] + ev
                return carry

            lax.fori_loop(0, D // _L, vec_body, 0)
            start_outs(t)

        for t in range(max(0, n_chunks - 3), n_chunks):
            wait_outs(t)

    return k


def kernel(x, emb):
    B, S, D = x.shape
    return _sc_add(B, S, D)(x, emb[:S])
